# baseline (speedup 1.0000x reference)
import functools

import jax
import jax.numpy as jnp
from jax import lax
from jax.experimental import pallas as pl
from jax.experimental.pallas import tpu as pltpu, tpu_sc as plsc

_B, _P, _D = 256, 1024, 96
_B_SC = 96
_ROWS_PER_BATCH = _D
_CHUNK = 16
_NW = 32
_BPW = _B_SC // _NW
_NPHASE = _ROWS_PER_BATCH // _CHUNK
_NSLOT = 4
_NCHUNK = _BPW * _NPHASE
_VPR = _P // 16

_CB_TC = 32


def _add_block(buf, slot, tbuf, tc):
    @plsc.parallel_loop(0, _CHUNK * _VPR, unroll=8)
    def _(k):
        j = k // _VPR
        sl = pl.ds((k % _VPR) * 16, 16)
        buf[slot, j, sl] = buf[slot, j, sl] + tbuf[tc, j, sl]


def _sc_kernel(x_hbm, t_hbm, o_hbm, tbuf, buf, tsems, insems, outsems):
    nc = 2
    wid = lax.axis_index("s") * nc + lax.axis_index("c")
    row0 = wid * _BPW * _ROWS_PER_BATCH

    def slot_of(c, b):
        return lax.rem(c * _BPW + b, _NSLOT)

    def t_copy(c, tc):
        return pltpu.make_async_copy(
            t_hbm.at[pl.ds(c * _CHUNK, _CHUNK)], tbuf.at[tc], tsems.at[tc]
        )

    def in_copy(c, b, slot):
        rows = pl.ds(row0 + b * _ROWS_PER_BATCH + c * _CHUNK, _CHUNK)
        return pltpu.make_async_copy(x_hbm.at[rows], buf.at[slot], insems.at[slot])

    def out_copy(c, b, slot):
        rows = pl.ds(row0 + b * _ROWS_PER_BATCH + c * _CHUNK, _CHUNK)
        return pltpu.make_async_copy(buf.at[slot], o_hbm.at[rows], outsems.at[slot])

    t_copy(0, 0).start()
    t_copy(1, 1).start()
    in_copy(0, 0, 0).start()
    in_copy(0, 1, 1).start()

    def phase(c, carry):
        tc = lax.rem(c, 2)
        for b in range(_BPW):
            slot = slot_of(c, b)
            in_copy(c, b, slot).wait()
            if b == 0:
                t_copy(c, tc).wait()
            _add_block(buf, slot, tbuf, tc)
            out_copy(c, b, slot).start()
            pb = (b - 2) % _BPW
            nb = (b + 2) % _BPW
            pslot = slot_of(c, b - 2)
            if b >= 2:
                out_copy(c, b - 2, pslot).wait()
                if b + 2 < _BPW:
                    in_copy(c, b + 2, slot_of(c, b + 2)).start()
                else:
                    @pl.when(c < _NPHASE - 1)
                    def _():
                        in_copy(c + 1, nb, slot_of(c + 1, nb)).start()
            else:
                @pl.when(c > 0)
                def _():
                    out_copy(c - 1, pb, pslot).wait()
                    if b + 2 < _BPW:
                        in_copy(c, b + 2, slot_of(c, b + 2)).start()
                    else:
                        @pl.when(c < _NPHASE - 1)
                        def _():
                            in_copy(c + 1, nb, slot_of(c + 1, nb)).start()

                if b + 2 >= _BPW:
                    @pl.when(c == 0)
                    def _():
                        in_copy(c + 1, nb, slot_of(c + 1, nb)).start()
                else:
                    @pl.when(c == 0)
                    def _():
                        in_copy(c, b + 2, slot_of(c, b + 2)).start()
            if b == _BPW - 1:
                @pl.when(c < _NPHASE - 2)
                def _():
                    t_copy(c + 2, tc).start()
        return carry

    lax.fori_loop(0, _NPHASE, phase, 0)
    out_copy(_NPHASE - 1, _BPW - 2, slot_of(_NPHASE - 1, _BPW - 2)).wait()
    out_copy(_NPHASE - 1, _BPW - 1, slot_of(_NPHASE - 1, _BPW - 1)).wait()


def _tc_body(x_ref, t_ref, o_ref):
    o_ref[...] = x_ref[...] + t_ref[...]


def kernel(encoded_patches, pos_table):
    B, P, D = encoded_patches.shape
    xt = jnp.swapaxes(encoded_patches, 1, 2)
    x2d = xt.reshape(B * D, P)
    t2d = pos_table.T

    mesh = plsc.VectorSubcoreMesh(core_axis_name="c", subcore_axis_name="s")
    sc_run = functools.partial(
        pl.kernel,
        mesh=mesh,
        out_type=jax.ShapeDtypeStruct((_B_SC * D, P), jnp.float32),
        scratch_types=[
            pltpu.VMEM((2, _CHUNK, _P), jnp.float32),
            pltpu.VMEM((_NSLOT, _CHUNK, _P), jnp.float32),
            pltpu.SemaphoreType.DMA((2,)),
            pltpu.SemaphoreType.DMA((_NSLOT,)),
            pltpu.SemaphoreType.DMA((_NSLOT,)),
        ],
    )(_sc_kernel)
    out_sc = sc_run(x2d, t2d)

    nb_sc = _B_SC // _CB_TC
    out_tc = pl.pallas_call(
        _tc_body,
        grid=((B - _B_SC) // _CB_TC,),
        in_specs=[
            pl.BlockSpec((_CB_TC, D, P), lambda i: (i + nb_sc, 0, 0)),
            pl.BlockSpec((D, P), lambda i: (0, 0)),
        ],
        out_specs=pl.BlockSpec((_CB_TC, D, P), lambda i: (i, 0, 0)),
        out_shape=jax.ShapeDtypeStruct((B - _B_SC, D, P), jnp.float32),
    )(xt, t2d)

    out_t = jnp.concatenate([out_sc.reshape(_B_SC, D, P), out_tc], axis=0)
    return jnp.swapaxes(out_t, 1, 2)


# speedup vs baseline: 1.3414x; 1.3414x over previous
import functools

import jax
import jax.numpy as jnp
from jax import lax
from jax.experimental import pallas as pl
from jax.experimental.pallas import tpu as pltpu, tpu_sc as plsc

_B, _P, _D = 256, 1024, 96
_ROWS_PER_BATCH = _D
_CHUNK = 16
_NW = 32
_BATCH_PER_W = _B // _NW
_NPHASE = _ROWS_PER_BATCH // _CHUNK
_NSLOT = 4
_VPR = _P // 16


def _add_block(buf, slot, tbuf, tc):
    @plsc.parallel_loop(0, _CHUNK * _VPR, unroll=8)
    def _(k):
        j = k // _VPR
        sl = pl.ds((k % _VPR) * 16, 16)
        buf[slot, j, sl] = buf[slot, j, sl] + tbuf[tc, j, sl]


def _sc_kernel(x_hbm, t_hbm, o_hbm, tbuf, buf, tsems, insems, outsems):
    nc = 2
    wid = lax.axis_index("s") * nc + lax.axis_index("c")
    row0 = wid * _BATCH_PER_W * _ROWS_PER_BATCH

    def t_copy(c, tc):
        return pltpu.make_async_copy(
            t_hbm.at[pl.ds(c * _CHUNK, _CHUNK)], tbuf.at[tc], tsems.at[tc]
        )

    def in_copy(c, b, slot):
        rows = pl.ds(row0 + b * _ROWS_PER_BATCH + c * _CHUNK, _CHUNK)
        return pltpu.make_async_copy(x_hbm.at[rows], buf.at[slot], insems.at[slot])

    def out_copy(c, b, slot):
        rows = pl.ds(row0 + b * _ROWS_PER_BATCH + c * _CHUNK, _CHUNK)
        return pltpu.make_async_copy(buf.at[slot], o_hbm.at[rows], outsems.at[slot])

    t_copy(0, 0).start()
    t_copy(1, 1).start()
    in_copy(0, 0, 0).start()
    in_copy(0, 1, 1).start()

    def phase(c, carry):
        tc = lax.rem(c, 2)
        for b in range(_BATCH_PER_W):
            slot = b % _NSLOT
            in_copy(c, b, slot).wait()
            if b == 0:
                t_copy(c, tc).wait()
            _add_block(buf, slot, tbuf, tc)
            out_copy(c, b, slot).start()
            if b >= 2:
                out_copy(c, b - 2, (b - 2) % _NSLOT).wait()
                if b + 2 < _BATCH_PER_W:
                    in_copy(c, b + 2, (b + 2) % _NSLOT).start()
                else:
                    @pl.when(c < _NPHASE - 1)
                    def _():
                        in_copy(c + 1, b + 2 - _BATCH_PER_W, (b + 2) % _NSLOT).start()
            else:
                @pl.when(c > 0)
                def _():
                    out_copy(c - 1, b + _BATCH_PER_W - 2, (b + 2) % _NSLOT).wait()
                    in_copy(c, b + 2, (b + 2) % _NSLOT).start()

                @pl.when(c == 0)
                def _():
                    in_copy(c, b + 2, (b + 2) % _NSLOT).start()
            if b == _BATCH_PER_W - 1:
                @pl.when(c < _NPHASE - 2)
                def _():
                    t_copy(c + 2, tc).start()
        return carry

    lax.fori_loop(0, _NPHASE, phase, 0)
    out_copy(_NPHASE - 1, _BATCH_PER_W - 2, (_BATCH_PER_W - 2) % _NSLOT).wait()
    out_copy(_NPHASE - 1, _BATCH_PER_W - 1, (_BATCH_PER_W - 1) % _NSLOT).wait()


def kernel(encoded_patches, pos_table):
    B, P, D = encoded_patches.shape
    xt = jnp.swapaxes(encoded_patches, 1, 2)
    x2d = xt.reshape(B * D, P)
    t2d = pos_table.T

    mesh = plsc.VectorSubcoreMesh(core_axis_name="c", subcore_axis_name="s")
    run = functools.partial(
        pl.kernel,
        mesh=mesh,
        out_type=jax.ShapeDtypeStruct((B * D, P), jnp.float32),
        scratch_types=[
            pltpu.VMEM((2, _CHUNK, _P), jnp.float32),
            pltpu.VMEM((_NSLOT, _CHUNK, _P), jnp.float32),
            pltpu.SemaphoreType.DMA((2,)),
            pltpu.SemaphoreType.DMA((_NSLOT,)),
            pltpu.SemaphoreType.DMA((_NSLOT,)),
        ],
    )(_sc_kernel)
    out2d = run(x2d, t2d)
    return jnp.swapaxes(out2d.reshape(B, D, P), 1, 2)
